# baseline (device time: 113016 ns/iter reference)
import jax
import jax.numpy as jnp
from jax import lax
from jax.experimental import pallas as pl
from jax.experimental.pallas import tpu as pltpu

N_DEV = 32
K_SUB = 4
S = N_DEV - 1

RING = [0, 8, 16, 24, 27, 19, 11, 3, 4, 12, 20, 28, 31, 23, 15, 7,
        6, 14, 22, 30, 29, 21, 13, 5, 2, 10, 18, 26, 25, 17, 9, 1]
assert sorted(RING) == list(range(32))


def kernel(x, w_mat):
    m, k_loc = x.shape
    _, n = w_mat.shape
    cm = m // N_DEV
    half = n // 2
    subw = half // K_SUB

    def body(x_ref, w_ref, ring_ref, out_ref, p_ref, acc_ref, recv_ref,
             send_sem, recv_sem):
        my = lax.axis_index("i")

        ring_arr = ring_ref[...]
        iota = lax.broadcasted_iota(jnp.int32, (1, N_DEV), 1)

        def ring_at(pos):
            p = lax.rem(pos + 4 * N_DEV, N_DEV)
            return jnp.sum(jnp.where(iota == p, ring_arr, 0))

        rp = jnp.sum(jnp.where(ring_arr == my, iota, 0))
        succ = ring_at(rp + 1)
        pred = ring_at(rp - 1)
        nbr_send = (succ, pred)

        barrier = pltpu.get_barrier_semaphore()
        for nbr in (pred, succ):
            pl.semaphore_signal(barrier, inc=1, device_id=(nbr,),
                                device_id_type=pl.DeviceIdType.MESH)
        pl.semaphore_wait(barrier, 2)

        def col0(d, j):
            return d * half + j * subw

        def chunk_row(d, s):
            if d == 0:
                return ring_at(rp - s - 1) * cm
            return ring_at(rp + s + 1) * cm

        def start_send(d, j, s):
            r = pltpu.make_async_remote_copy(
                src_ref=acc_ref.at[d, j, s],
                dst_ref=recv_ref.at[d, j, s],
                send_sem=send_sem.at[d, j, s],
                recv_sem=recv_sem.at[d, j, s],
                device_id=(nbr_send[d],),
                device_id_type=pl.DeviceIdType.MESH,
            )
            r.start()
            return r

        chains = [(d, j) for d in range(2) for j in range(K_SUB)]
        rdmas = {}

        for d in range(2):
            p_ref[:, d * half:(d + 1) * half] = jnp.dot(
                x_ref[...], w_ref[:, d * half:(d + 1) * half],
                preferred_element_type=jnp.float32,
            )
            row = chunk_row(d, 0)
            for j in range(K_SUB):
                acc_ref[d, j, 0, :, :] = p_ref[
                    pl.ds(row, cm), col0(d, j):col0(d, j) + subw
                ]
                rdmas[(d, j, 0)] = start_send(d, j, 0)

        for s in range(S):
            if s < S - 1:
                rows = (chunk_row(0, s + 1), chunk_row(1, s + 1))
            else:
                rows = (my * cm, my * cm)
            for d, j in chains:
                rdmas[(d, j, s)].wait_recv()
                psl = p_ref[pl.ds(rows[d], cm), col0(d, j):col0(d, j) + subw]
                if s < S - 1:
                    acc_ref[d, j, s + 1, :, :] = (
                        recv_ref[d, j, s, :, :] + psl
                    )
                    rdmas[(d, j, s + 1)] = start_send(d, j, s + 1)
                else:
                    out_ref[:, col0(d, j):col0(d, j) + subw] = jnp.maximum(
                        recv_ref[d, j, s, :, :] + psl, 0.0
                    )

        for d, j in chains:
            for s in range(S):
                rdmas[(d, j, s)].wait_send()

    return pl.pallas_call(
        body,
        out_shape=jax.ShapeDtypeStruct((cm, n), jnp.float32),
        in_specs=[
            pl.BlockSpec(memory_space=pltpu.VMEM),
            pl.BlockSpec(memory_space=pltpu.VMEM),
            pl.BlockSpec(memory_space=pltpu.VMEM),
        ],
        out_specs=pl.BlockSpec(memory_space=pltpu.VMEM),
        scratch_shapes=[
            pltpu.VMEM((m, n), jnp.float32),
            pltpu.VMEM((2, K_SUB, S, cm, subw), jnp.float32),
            pltpu.VMEM((2, K_SUB, S, cm, subw), jnp.float32),
            pltpu.SemaphoreType.DMA((2, K_SUB, S)),
            pltpu.SemaphoreType.DMA((2, K_SUB, S)),
        ],
        compiler_params=pltpu.CompilerParams(
            collective_id=0, vmem_limit_bytes=100 * 1024 * 1024,
        ),
    )(x, w_mat, jnp.asarray(RING, dtype=jnp.int32).reshape(1, N_DEV))
